# baseline (speedup 1.0000x reference)
import functools

import jax
import jax.numpy as jnp
from jax import lax
from jax.experimental import pallas as pl
from jax.experimental.pallas import tpu as pltpu
from jax.experimental.pallas import tpu_sc as plsc

_DT = 8e-08
_VS = 1500.0
_NX = 256
_NY = 256
_DX = 0.001
_DY = 0.001

_B = 8
_S = 128
_T = 16384

_NW = 32
_G = 1
_GPAIR = (_S // 2) // _G
_WPAIR = _GPAIR // 4
_CROWS = 32
_NCHUNK = _NY // _CROWS


def _tmap_body(g, mask_ref, out_ref):
    b = pl.program_id(0)
    p = pl.program_id(1)
    s2 = (g * _GPAIR + p) * 2
    col = lax.broadcasted_iota(jnp.int32, (1, _NX), 1).astype(jnp.float32) + 1.0
    row = lax.broadcasted_iota(jnp.int32, (_NY, 1), 0).astype(jnp.float32) + 1.0

    def tmap(s):
        x = mask_ref[b, s, 0] * 1000.0 + 128.0
        y = mask_ref[b, s, 1] * 1000.0 + 128.0
        dx = (x - col + 1.0) * _DX
        dy = (y - row + 1.0) * _DY
        dis = jnp.sqrt(dx * dx + dy * dy)
        return (dis * jnp.float32(8333.33301)).astype(jnp.int32)

    ta = tmap(s2)
    tb = tmap(s2 + 1)
    out_ref[0, 0] = ta | ((tb + _T) << 16)


def _tmap_call(sensor_mask, g):
    return pl.pallas_call(
        functools.partial(_tmap_body, g),
        grid=(_B, _GPAIR),
        in_specs=[pl.BlockSpec(memory_space=pltpu.SMEM)],
        out_specs=pl.BlockSpec((1, 1, _NY, _NX), lambda b, p: (b, p, 0, 0)),
        out_shape=jax.ShapeDtypeStruct((_B, _GPAIR, _NY, _NX), jnp.int32),
    )(sensor_mask)


_sc_mesh = plsc.VectorSubcoreMesh(core_axis_name="c", subcore_axis_name="s")


def _sc_body(g, data_hbm, tmap_hbm, out_hbm, pair_v, idx0_v, idx1_v, acc_v,
             sem0, sem1):
    cid = lax.axis_index("c")
    sid = lax.axis_index("s")
    wid = sid * 2 + cid
    b = wid // 4
    grp = wid % 4
    row0 = grp * _WPAIR

    idx_bufs = (idx0_v, idx1_v)
    sems = (sem0, sem1)

    zero = jnp.zeros((16,), jnp.float32)

    @plsc.parallel_loop(0, _NY * _NX, step=16, unroll=8)
    def _zero_loop(i):
        r = lax.shift_right_logical(i, 8)
        col = i & jnp.int32(_NX - 1)
        acc_v[r, pl.ds(col, 16)] = zero

    pltpu.async_copy(tmap_hbm.at[b, row0, pl.ds(0, _CROWS), :], idx0_v, sem0)

    def pair_body(p, carry):
        prow = row0 + p
        s2 = (g * _GPAIR + prow) * 2
        pltpu.sync_copy(data_hbm.at[b, s2], pair_v.at[pl.ds(0, _T)])
        pltpu.sync_copy(data_hbm.at[b, s2 + 1], pair_v.at[pl.ds(_T, _T)])

        for c in range(_NCHUNK):
            buf = idx_bufs[c % 2]
            sem = sems[c % 2]
            nbuf = idx_bufs[(c + 1) % 2]
            nsem = sems[(c + 1) % 2]
            pltpu.make_async_copy(
                tmap_hbm.at[b, prow, pl.ds(0, _CROWS), :], buf, sem).wait()
            if c + 1 < _NCHUNK:
                nrow, noff = prow, (c + 1) * _CROWS
            else:
                nrow, noff = jnp.minimum(prow + 1, row0 + _WPAIR - 1), 0
            pltpu.async_copy(
                tmap_hbm.at[b, nrow, pl.ds(noff, _CROWS), :], nbuf, nsem)

            base_row = c * _CROWS

            @plsc.parallel_loop(0, _CROWS * _NX, step=16, unroll=8)
            def _gather_loop(i):
                r = lax.shift_right_logical(i, 8)
                col = i & jnp.int32(_NX - 1)
                iv = buf[r, pl.ds(col, 16)]
                ia = iv & jnp.int32(0xFFFF)
                ib = lax.shift_right_logical(iv, 16)
                ga = plsc.load_gather(pair_v, [ia])
                gb = plsc.load_gather(pair_v, [ib])
                plsc.addupdate(acc_v.at[base_row + r, pl.ds(col, 16)], ga + gb)

        return carry

    lax.fori_loop(0, _WPAIR, pair_body, 0)
    pltpu.make_async_copy(
        tmap_hbm.at[b, row0, pl.ds(0, _CROWS), :], idx_bufs[0], sems[0]).wait()
    pltpu.sync_copy(acc_v, out_hbm.at[b, grp])


def _make_sc(g):
    return functools.partial(
        pl.kernel,
        mesh=_sc_mesh,
        out_type=jax.ShapeDtypeStruct((_B, 4, _NY, _NX), jnp.float32),
        scratch_types=[
            pltpu.VMEM((2 * _T,), jnp.float32),
            pltpu.VMEM((_CROWS, _NX), jnp.int32),
            pltpu.VMEM((_CROWS, _NX), jnp.int32),
            pltpu.VMEM((_NY, _NX), jnp.float32),
            pltpu.SemaphoreType.DMA,
            pltpu.SemaphoreType.DMA,
        ],
        compiler_params=pltpu.CompilerParams(needs_layout_passes=False),
    )(functools.partial(_sc_body, g))


_sc_calls = [_make_sc(g) for g in range(_G)]


def _norm_body(*refs):
    part_refs, out_ref = refs[:-1], refs[-1]

    def s4(ref):
        p = ref[0]
        return (p[0] + p[1]) + (p[2] + p[3])

    imgs = [s4(r) for r in part_refs]
    while len(imgs) > 1:
        imgs = [a + b for a, b in zip(imgs[::2], imgs[1::2])]
    img = imgs[0]
    mn = jnp.min(img)
    mx = jnp.max(img)
    out_ref[0] = (img - mn) / (mx - mn)


def _norm_call(parts):
    spec = pl.BlockSpec((1, 4, _NY, _NX), lambda b: (b, 0, 0, 0))
    return pl.pallas_call(
        _norm_body,
        grid=(_B,),
        in_specs=[spec] * _G,
        out_specs=pl.BlockSpec((1, _NY, _NX), lambda b: (b, 0, 0)),
        out_shape=jax.ShapeDtypeStruct((_B, _NY, _NX), jnp.float32),
    )(*parts)


def kernel(sensor_data, sensor_mask):
    parts = []
    for g in range(_G):
        tm = _tmap_call(sensor_mask, g)
        parts.append(_sc_calls[g](sensor_data, tm))
    return _norm_call(parts)


# speedup vs baseline: 1.2225x; 1.2225x over previous
import functools

import jax
import jax.numpy as jnp
from jax import lax
from jax.experimental import pallas as pl
from jax.experimental.pallas import tpu as pltpu
from jax.experimental.pallas import tpu_sc as plsc

_DT = 8e-08
_VS = 1500.0
_NX = 256
_NY = 256
_DX = 0.001
_DY = 0.001

_B = 8
_S = 128
_T = 16384

_NW = 32
_G = 8
_GPAIR = (_S // 2) // _G
_WPAIR = _GPAIR // 4
_CROWS = 32
_NCHUNK = _NY // _CROWS


def _tmap_body(g, mask_ref, out_ref):
    b = pl.program_id(0)
    p = pl.program_id(1)
    s2 = (g * _GPAIR + p) * 2
    col = lax.broadcasted_iota(jnp.int32, (1, _NX), 1).astype(jnp.float32) + 1.0
    row = lax.broadcasted_iota(jnp.int32, (_NY, 1), 0).astype(jnp.float32) + 1.0

    def tmap(s):
        x = mask_ref[b, s, 0] * 1000.0 + 128.0
        y = mask_ref[b, s, 1] * 1000.0 + 128.0
        dx = (x - col + 1.0) * _DX
        dy = (y - row + 1.0) * _DY
        dis = jnp.sqrt(dx * dx + dy * dy)
        return (dis * jnp.float32(8333.33301)).astype(jnp.int32)

    ta = tmap(s2)
    tb = tmap(s2 + 1)
    out_ref[0, 0] = ta | ((tb + _T) << 16)


def _tmap_call(sensor_mask, g):
    return pl.pallas_call(
        functools.partial(_tmap_body, g),
        grid=(_B, _GPAIR),
        in_specs=[pl.BlockSpec(memory_space=pltpu.SMEM)],
        out_specs=pl.BlockSpec((1, 1, _NY, _NX), lambda b, p: (b, p, 0, 0)),
        out_shape=jax.ShapeDtypeStruct((_B, _GPAIR, _NY, _NX), jnp.int32),
    )(sensor_mask)


_sc_mesh = plsc.VectorSubcoreMesh(core_axis_name="c", subcore_axis_name="s")


def _sc_body(g, data_hbm, tmap_hbm, out_hbm, pair_v, idx0_v, idx1_v, acc_v,
             sem0, sem1):
    cid = lax.axis_index("c")
    sid = lax.axis_index("s")
    wid = sid * 2 + cid
    b = wid // 4
    grp = wid % 4
    row0 = grp * _WPAIR

    idx_bufs = (idx0_v, idx1_v)
    sems = (sem0, sem1)

    zero = jnp.zeros((16,), jnp.float32)

    @plsc.parallel_loop(0, _NY * _NX, step=16, unroll=8)
    def _zero_loop(i):
        r = lax.shift_right_logical(i, 8)
        col = i & jnp.int32(_NX - 1)
        acc_v[r, pl.ds(col, 16)] = zero

    pltpu.async_copy(tmap_hbm.at[b, row0, pl.ds(0, _CROWS), :], idx0_v, sem0)

    def pair_body(p, carry):
        prow = row0 + p
        s2 = (g * _GPAIR + prow) * 2
        pltpu.sync_copy(data_hbm.at[b, s2], pair_v.at[pl.ds(0, _T)])
        pltpu.sync_copy(data_hbm.at[b, s2 + 1], pair_v.at[pl.ds(_T, _T)])

        for c in range(_NCHUNK):
            buf = idx_bufs[c % 2]
            sem = sems[c % 2]
            nbuf = idx_bufs[(c + 1) % 2]
            nsem = sems[(c + 1) % 2]
            pltpu.make_async_copy(
                tmap_hbm.at[b, prow, pl.ds(0, _CROWS), :], buf, sem).wait()
            if c + 1 < _NCHUNK:
                nrow, noff = prow, (c + 1) * _CROWS
            else:
                nrow, noff = jnp.minimum(prow + 1, row0 + _WPAIR - 1), 0
            pltpu.async_copy(
                tmap_hbm.at[b, nrow, pl.ds(noff, _CROWS), :], nbuf, nsem)

            base_row = c * _CROWS

            @plsc.parallel_loop(0, _CROWS * _NX, step=16, unroll=8)
            def _gather_loop(i):
                r = lax.shift_right_logical(i, 8)
                col = i & jnp.int32(_NX - 1)
                iv = buf[r, pl.ds(col, 16)]
                ia = iv & jnp.int32(0xFFFF)
                ib = lax.shift_right_logical(iv, 16)
                ga = plsc.load_gather(pair_v, [ia])
                gb = plsc.load_gather(pair_v, [ib])
                plsc.addupdate(acc_v.at[base_row + r, pl.ds(col, 16)], ga + gb)

        return carry

    lax.fori_loop(0, _WPAIR, pair_body, 0)
    pltpu.make_async_copy(
        tmap_hbm.at[b, row0, pl.ds(0, _CROWS), :], idx_bufs[0], sems[0]).wait()
    pltpu.sync_copy(acc_v, out_hbm.at[b, grp])


def _make_sc(g):
    return functools.partial(
        pl.kernel,
        mesh=_sc_mesh,
        out_type=jax.ShapeDtypeStruct((_B, 4, _NY, _NX), jnp.float32),
        scratch_types=[
            pltpu.VMEM((2 * _T,), jnp.float32),
            pltpu.VMEM((_CROWS, _NX), jnp.int32),
            pltpu.VMEM((_CROWS, _NX), jnp.int32),
            pltpu.VMEM((_NY, _NX), jnp.float32),
            pltpu.SemaphoreType.DMA,
            pltpu.SemaphoreType.DMA,
        ],
        compiler_params=pltpu.CompilerParams(needs_layout_passes=False),
    )(functools.partial(_sc_body, g))


_sc_calls = [_make_sc(g) for g in range(_G)]


def _norm_body(*refs):
    part_refs, out_ref = refs[:-1], refs[-1]

    def s4(ref):
        p = ref[0]
        return (p[0] + p[1]) + (p[2] + p[3])

    imgs = [s4(r) for r in part_refs]
    while len(imgs) > 1:
        imgs = [a + b for a, b in zip(imgs[::2], imgs[1::2])]
    img = imgs[0]
    mn = jnp.min(img)
    mx = jnp.max(img)
    out_ref[0] = (img - mn) / (mx - mn)


def _norm_call(parts):
    spec = pl.BlockSpec((1, 4, _NY, _NX), lambda b: (b, 0, 0, 0))
    return pl.pallas_call(
        _norm_body,
        grid=(_B,),
        in_specs=[spec] * _G,
        out_specs=pl.BlockSpec((1, _NY, _NX), lambda b: (b, 0, 0)),
        out_shape=jax.ShapeDtypeStruct((_B, _NY, _NX), jnp.float32),
    )(*parts)


def kernel(sensor_data, sensor_mask):
    parts = []
    for g in range(_G):
        tm = _tmap_call(sensor_mask, g)
        parts.append(_sc_calls[g](sensor_data, tm))
    return _norm_call(parts)


# speedup vs baseline: 1.3442x; 1.0995x over previous
import functools

import jax
import jax.numpy as jnp
from jax import lax
from jax.experimental import pallas as pl
from jax.experimental.pallas import tpu as pltpu
from jax.experimental.pallas import tpu_sc as plsc

_DT = 8e-08
_VS = 1500.0
_NX = 256
_NY = 256
_DX = 0.001
_DY = 0.001

_B = 8
_S = 128
_T = 16384

_NW = 32
_GROUPS = [(0, 8), (8, 20), (28, 20), (48, 16)]
_CROWS = 32
_NCHUNK = _NY // _CROWS


def _tmap_body(start, mask_ref, out_ref):
    b = pl.program_id(0)
    p = pl.program_id(1)
    s2 = (start + p) * 2
    col = lax.broadcasted_iota(jnp.int32, (1, _NX), 1).astype(jnp.float32) + 1.0
    row = lax.broadcasted_iota(jnp.int32, (_NY, 1), 0).astype(jnp.float32) + 1.0

    def tmap(s):
        x = mask_ref[b, s, 0] * 1000.0 + 128.0
        y = mask_ref[b, s, 1] * 1000.0 + 128.0
        dx = (x - col + 1.0) * _DX
        dy = (y - row + 1.0) * _DY
        dis = jnp.sqrt(dx * dx + dy * dy)
        return (dis * jnp.float32(8333.33301)).astype(jnp.int32)

    ta = tmap(s2)
    tb = tmap(s2 + 1)
    out_ref[0, 0] = ta | ((tb + _T) << 16)


def _tmap_call(sensor_mask, start, npair):
    return pl.pallas_call(
        functools.partial(_tmap_body, start),
        grid=(_B, npair),
        in_specs=[pl.BlockSpec(memory_space=pltpu.SMEM)],
        out_specs=pl.BlockSpec((1, 1, _NY, _NX), lambda b, p: (b, p, 0, 0)),
        out_shape=jax.ShapeDtypeStruct((_B, npair, _NY, _NX), jnp.int32),
    )(sensor_mask)


_sc_mesh = plsc.VectorSubcoreMesh(core_axis_name="c", subcore_axis_name="s")


def _sc_body(start, npair, data_hbm, tmap_hbm, out_hbm, pair_v, idx0_v, idx1_v,
             acc_v, sem0, sem1):
    wp = npair // 4
    cid = lax.axis_index("c")
    sid = lax.axis_index("s")
    wid = sid * 2 + cid
    b = wid // 4
    grp = wid % 4
    row0 = grp * wp

    idx_bufs = (idx0_v, idx1_v)
    sems = (sem0, sem1)

    pltpu.async_copy(tmap_hbm.at[b, row0, pl.ds(0, _CROWS), :], idx0_v, sem0)

    def emit_pair(prow, first):
        s2 = (start + prow) * 2
        pltpu.sync_copy(data_hbm.at[b, s2], pair_v.at[pl.ds(0, _T)])
        pltpu.sync_copy(data_hbm.at[b, s2 + 1], pair_v.at[pl.ds(_T, _T)])

        for c in range(_NCHUNK):
            buf = idx_bufs[c % 2]
            sem = sems[c % 2]
            nbuf = idx_bufs[(c + 1) % 2]
            nsem = sems[(c + 1) % 2]
            pltpu.make_async_copy(
                tmap_hbm.at[b, prow, pl.ds(0, _CROWS), :], buf, sem).wait()
            if c + 1 < _NCHUNK:
                nrow, noff = prow, (c + 1) * _CROWS
            else:
                nrow, noff = jnp.minimum(prow + 1, row0 + wp - 1), 0
            pltpu.async_copy(
                tmap_hbm.at[b, nrow, pl.ds(noff, _CROWS), :], nbuf, nsem)

            base_row = c * _CROWS

            @plsc.parallel_loop(0, _CROWS * _NX, step=16, unroll=8)
            def _gather_loop(i):
                r = lax.shift_right_logical(i, 8)
                col = i & jnp.int32(_NX - 1)
                iv = buf[r, pl.ds(col, 16)]
                ia = iv & jnp.int32(0xFFFF)
                ib = lax.shift_right_logical(iv, 16)
                ga = plsc.load_gather(pair_v, [ia])
                gb = plsc.load_gather(pair_v, [ib])
                if first:
                    acc_v[base_row + r, pl.ds(col, 16)] = ga + gb
                else:
                    plsc.addupdate(acc_v.at[base_row + r, pl.ds(col, 16)], ga + gb)

    emit_pair(row0, True)

    def pair_body(p, carry):
        emit_pair(row0 + p, False)
        return carry

    lax.fori_loop(1, wp, pair_body, 0)
    pltpu.make_async_copy(
        tmap_hbm.at[b, row0, pl.ds(0, _CROWS), :], idx_bufs[0], sems[0]).wait()
    pltpu.sync_copy(acc_v, out_hbm.at[b, grp])


def _make_sc(start, npair):
    return functools.partial(
        pl.kernel,
        mesh=_sc_mesh,
        out_type=jax.ShapeDtypeStruct((_B, 4, _NY, _NX), jnp.float32),
        scratch_types=[
            pltpu.VMEM((2 * _T,), jnp.float32),
            pltpu.VMEM((_CROWS, _NX), jnp.int32),
            pltpu.VMEM((_CROWS, _NX), jnp.int32),
            pltpu.VMEM((_NY, _NX), jnp.float32),
            pltpu.SemaphoreType.DMA,
            pltpu.SemaphoreType.DMA,
        ],
        compiler_params=pltpu.CompilerParams(needs_layout_passes=False),
    )(functools.partial(_sc_body, start, npair))


_sc_calls = [_make_sc(start, npair) for start, npair in _GROUPS]


def _norm_body(*refs):
    part_refs, out_ref = refs[:-1], refs[-1]

    def s4(ref):
        p = ref[0]
        return (p[0] + p[1]) + (p[2] + p[3])

    imgs = [s4(r) for r in part_refs]
    while len(imgs) > 1:
        imgs = [a + b for a, b in zip(imgs[::2], imgs[1::2])]
    img = imgs[0]
    mn = jnp.min(img)
    mx = jnp.max(img)
    out_ref[0] = (img - mn) / (mx - mn)


def _norm_call(parts):
    spec = pl.BlockSpec((1, 4, _NY, _NX), lambda b: (b, 0, 0, 0))
    return pl.pallas_call(
        _norm_body,
        grid=(_B,),
        in_specs=[spec] * len(parts),
        out_specs=pl.BlockSpec((1, _NY, _NX), lambda b: (b, 0, 0)),
        out_shape=jax.ShapeDtypeStruct((_B, _NY, _NX), jnp.float32),
    )(*parts)


def kernel(sensor_data, sensor_mask):
    parts = []
    for g, (start, npair) in enumerate(_GROUPS):
        tm = _tmap_call(sensor_mask, start, npair)
        parts.append(_sc_calls[g](sensor_data, tm))
    return _norm_call(parts)
